# baseline (speedup 1.0000x reference)
import jax
import jax.numpy as jnp
from jax import lax
from jax.experimental import pallas as pl
from jax.experimental.pallas import tpu as pltpu
from jax.experimental.pallas import tpu_sc as plsc

NUM_CORES = 2
NUM_SUBCORES = 16
NW = NUM_CORES * NUM_SUBCORES

T = 26
HIDDEN = 128
CHUNK = 128
N_TOTAL = 100000
NUM_CHUNKS = (N_TOTAL + CHUNK - 1) // CHUNK
BIG_WORKERS = NUM_CHUNKS - 24 * NW
MAIN_CHUNKS = 24
SLOTS_PER_WORKER = 25 * CHUNK
NP = 102400
LAST_BASE = N_TOTAL - CHUNK


def _combine_body(w0_ref, w1_ref, w2_ref, xt_ref, s01_ref, c01_ref, c2_ref):
    w0, w1 = w0_ref[...], w1_ref[...]
    s01_ref[...] = (w0[:, None, :] + w1[None, :, :]).reshape(T * T, HIDDEN)
    xt = xt_ref[...]
    c01_ref[...] = T * xt[0] + xt[1]
    c2_ref[...] = xt[2]
    del w2_ref


@jax.jit
def _combine(w0, w1, w2, xt3):
    return pl.pallas_call(
        _combine_body,
        out_shape=(
            jax.ShapeDtypeStruct((T * T, HIDDEN), jnp.float32),
            jax.ShapeDtypeStruct(xt3.shape[1:], jnp.int32),
            jax.ShapeDtypeStruct(xt3.shape[1:], jnp.int32),
        ),
    )(w0, w1, w2, xt3)


def _sc_body(c01_hbm, c2_hbm, w2_hbm, s01_hbm, out_hbm,
             idx01_v, idx2_v, buf0, buf1, buf2, buf3, s01_sp, w2_sp,
             sem_a, sem_g0, sem_g1, sem_g2, sem_g3,
             sem_w0, sem_w1, sem_w2, sem_w3):
    core = lax.axis_index("c")
    sid = lax.axis_index("s")
    wid = sid * NUM_CORES + core
    start = wid * MAIN_CHUNKS + jnp.minimum(wid, BIG_WORKERS)
    sbase = pl.multiple_of(start * CHUNK, CHUNK)

    @pl.when(sid == 0)
    def _():
        pltpu.sync_copy(s01_hbm, s01_sp)
        pltpu.sync_copy(w2_hbm, w2_sp)

    pltpu.sync_copy(c01_hbm.at[pl.ds(sbase, SLOTS_PER_WORKER)], idx01_v)
    pltpu.sync_copy(c2_hbm.at[pl.ds(sbase, SLOTS_PER_WORKER)], idx2_v)
    plsc.subcore_barrier()

    bufs = (buf0, buf1, buf2, buf3)
    gsems = (sem_g0, sem_g1, sem_g2, sem_g3)
    wsems = (sem_w0, sem_w1, sem_w2, sem_w3)
    NBUF = 4

    def chunk_base(c):
        ob = jnp.minimum((start + c) * CHUNK, LAST_BASE)
        return pl.multiple_of(ob, 32)

    def idx_off(c):
        off = jnp.minimum(chunk_base(c) - sbase, SLOTS_PER_WORKER - CHUNK)
        return pl.multiple_of(off, 32)

    def gather01(c, s):
        pltpu.async_copy(
            s01_sp.at[idx01_v.at[pl.ds(idx_off(c), CHUNK)]], bufs[s],
            gsems[s])

    def wait_g01(s):
        pltpu.make_async_copy(
            out_hbm.at[pl.ds(0, CHUNK), :], bufs[s], gsems[s]).wait()

    def gather2_add(c, s):
        return pltpu.async_copy(
            w2_sp.at[idx2_v.at[pl.ds(idx_off(c), CHUNK)]], bufs[s],
            sem_a, add=True)

    def write(c, s):
        pltpu.async_copy(
            bufs[s], out_hbm.at[pl.ds(chunk_base(c), CHUNK), :], wsems[s])

    def wait_write(s):
        pltpu.make_async_copy(
            bufs[s], out_hbm.at[pl.ds(0, CHUNK), :], wsems[s]).wait()

    def step(c, s, wait_prev_write):
        wait_g01(s)
        h_a = gather2_add(c, s)
        nxt = (s + 2) % NBUF
        if wait_prev_write:
            wait_write(nxt)
        gather01(c + 2, nxt)
        h_a.wait()
        write(c, s)

    gather01(0, 0)
    gather01(1, 1)
    step(0, 0, False)
    step(1, 1, False)
    step(2, 2, True)
    step(3, 3, True)

    def body(k, carry):
        c = 4 * k
        step(c, 0, True)
        step(c + 1, 1, True)
        step(c + 2, 2, True)
        step(c + 3, 3, True)
        return carry

    lax.fori_loop(1, 6, body, 0)

    wait_g01(0)

    @pl.when(wid < BIG_WORKERS)
    def _():
        h_a = gather2_add(24, 0)
        h_a.wait()
        write(24, 0)
        wait_write(0)

    wait_g01(1)
    wait_write(2)
    wait_write(3)


@jax.jit
def _encode(c01, c2, w2, s01):
    mesh = plsc.VectorSubcoreMesh(core_axis_name="c", subcore_axis_name="s")
    return pl.kernel(
        _sc_body,
        out_type=jax.ShapeDtypeStruct((N_TOTAL, HIDDEN), jnp.float32),
        mesh=mesh,
        scratch_types=[
            pltpu.VMEM((SLOTS_PER_WORKER,), jnp.int32),
            pltpu.VMEM((SLOTS_PER_WORKER,), jnp.int32),
            pltpu.VMEM((CHUNK, HIDDEN), jnp.float32),
            pltpu.VMEM((CHUNK, HIDDEN), jnp.float32),
            pltpu.VMEM((CHUNK, HIDDEN), jnp.float32),
            pltpu.VMEM((CHUNK, HIDDEN), jnp.float32),
            pltpu.VMEM_SHARED((T * T, HIDDEN), jnp.float32),
            pltpu.VMEM_SHARED((T, HIDDEN), jnp.float32),
        ] + [pltpu.SemaphoreType.DMA] * 9,
    )(c01, c2, w2, s01)


def kernel(x, W0, W1, W2):
    if x.ndim == 1:
        x = x[:, None]
    n = x.shape[0]
    xt = jnp.pad(x.T.astype(jnp.int32), ((0, 0), (0, NP - n)))
    s01, c01, c2 = _combine(W0, W1, W2, xt.reshape(3, NP // HIDDEN, HIDDEN))
    return _encode(c01.reshape(NP), c2.reshape(NP), W2, s01)
